# Initial kernel scaffold; baseline (speedup 1.0000x reference)
import jax
import jax.numpy as jnp
from jax.experimental import pallas as pl


def kernel(day, week, month, day_table, week_table, month_table):
    raise NotImplementedError("write your pallas kernel here")


# speedup vs baseline: 20.5113x; 20.5113x over previous
import functools

import jax
import jax.numpy as jnp
from jax import lax
from jax.experimental import pallas as pl
from jax.experimental.pallas import tpu as pltpu
from jax.experimental.pallas import tpu_sc as plsc

NC, NS = 2, 16
NW = NC * NS
B, L, D = 4096, 200, 128
N = B * L
ROWS_PER_W = N // NW
CHUNK = 128
NCHUNK = ROWS_PER_W // CHUNK
NDAY, NWEEK, NMONTH = 32, 7, 13
WM = NWEEK * NMONTH
COMB = NDAY * WM

_mesh = plsc.VectorSubcoreMesh(
    core_axis_name="c", subcore_axis_name="s", num_cores=NC, num_subcores=NS)


@functools.partial(
    pl.kernel,
    out_type=jax.ShapeDtypeStruct((COMB * D,), jnp.float32),
    mesh=_mesh,
    scratch_types=[
        pltpu.VMEM((D,), jnp.float32),
        pltpu.VMEM((NWEEK * D,), jnp.float32),
        pltpu.VMEM((NMONTH * D,), jnp.float32),
        pltpu.VMEM((WM * D,), jnp.float32),
    ],
)
def _build_comb(dt_h, wt_h, mt_h, comb_h, drow_v, wt_v, mt_v, buf_v):
    wid = lax.axis_index("s") * NC + lax.axis_index("c")
    pltpu.sync_copy(dt_h.at[pl.ds(wid * D, D)], drow_v)
    pltpu.sync_copy(wt_h, wt_v)
    pltpu.sync_copy(mt_h, mt_v)
    for w in range(NWEEK):
        dw = [drow_v[pl.ds(j * 16, 16)] + wt_v[pl.ds(w * D + j * 16, 16)]
              for j in range(D // 16)]
        for m in range(NMONTH):
            o = (w * NMONTH + m) * D
            for j in range(D // 16):
                buf_v[pl.ds(o + j * 16, 16)] = dw[j] + mt_v[pl.ds(m * D + j * 16, 16)]
    pltpu.sync_copy(buf_v, comb_h.at[pl.ds(wid * WM * D, WM * D)])


IDXBLK = 2560
NBLK = ROWS_PER_W // IDXBLK
GCH = 128
NCH = ROWS_PER_W // GCH


@functools.partial(
    pl.kernel,
    out_type=jax.ShapeDtypeStruct((N, D), jnp.float32),
    mesh=_mesh,
    scratch_types=[
        pltpu.VMEM((IDXBLK,), jnp.int32),
        pltpu.VMEM((IDXBLK,), jnp.int32),
        pltpu.VMEM((IDXBLK,), jnp.int32),
        pltpu.VMEM((ROWS_PER_W // 128, 128), jnp.int32),
        pltpu.VMEM((GCH, D), jnp.float32),
        pltpu.VMEM((GCH, D), jnp.float32),
        pltpu.SemaphoreType.DMA,
        pltpu.SemaphoreType.DMA,
        pltpu.SemaphoreType.DMA,
        pltpu.SemaphoreType.DMA,
    ],
)
def _lookup(day_h, week_h, month_h, comb_h, out_h,
            di, wi, mi, ci, rows0, rows1, gs0, gs1, ws0, ws1):
    wid = lax.axis_index("s") * NC + lax.axis_index("c")
    base0 = wid * ROWS_PER_W

    def blk(b, carry):
        boff = base0 + b * IDXBLK
        pltpu.sync_copy(day_h.at[pl.ds(boff, IDXBLK)], di)
        pltpu.sync_copy(week_h.at[pl.ds(boff, IDXBLK)], wi)
        pltpu.sync_copy(month_h.at[pl.ds(boff, IDXBLK)], mi)
        for g in range(IDXBLK // 16):
            s = pl.ds(g * 16, 16)
            d = jnp.clip(di[s], 0, NDAY - 1)
            w = jnp.clip(wi[s], 0, NWEEK - 1)
            m = jnp.clip(mi[s], 0, NMONTH - 1)
            ci[b * (IDXBLK // 128) + g // 8, pl.ds((g % 8) * 16, 16)] = (
                d * WM + w * NMONTH + m)
        return carry

    lax.fori_loop(0, NBLK, blk, 0)

    def gather(c, rbuf, sem):
        return pltpu.make_async_copy(comb_h.at[ci.at[c]], rbuf, sem)

    def write(c, rbuf, sem):
        return pltpu.make_async_copy(
            rbuf, out_h.at[pl.ds(base0 + c * GCH, GCH)], sem)

    gather(0, rows0, gs0).start()

    def step(k2, carry):
        c0 = 2 * k2
        c1 = c0 + 1

        @pl.when(k2 > 0)
        def _():
            write(c1 - 2, rows1, ws1).wait()
        gather(c1, rows1, gs1).start()
        gather(c0, rows0, gs0).wait()
        write(c0, rows0, ws0).start()

        @pl.when(k2 < NCH // 2 - 1)
        def _():
            write(c0, rows0, ws0).wait()
            gather(c0 + 2, rows0, gs0).start()
        gather(c1, rows1, gs1).wait()
        write(c1, rows1, ws1).start()
        return carry

    lax.fori_loop(0, NCH // 2, step, 0)
    write(NCH - 2, rows0, ws0).wait()
    write(NCH - 1, rows1, ws1).wait()


def kernel(day, week, month, day_table, week_table, month_table):
    day = day.reshape(N).astype(jnp.int32)
    week = week.reshape(N).astype(jnp.int32)
    month = month.reshape(N).astype(jnp.int32)
    comb = _build_comb(day_table.reshape(-1), week_table.reshape(-1),
                       month_table.reshape(-1)).reshape(COMB, D)
    out = _lookup(day, week, month, comb)
    return out.reshape(B, L, D)


# speedup vs baseline: 20.7541x; 1.0118x over previous
import functools

import jax
import jax.numpy as jnp
from jax import lax
from jax.experimental import pallas as pl
from jax.experimental.pallas import tpu as pltpu
from jax.experimental.pallas import tpu_sc as plsc

NC, NS = 2, 16
NW = NC * NS
B, L, D = 4096, 200, 128
N = B * L
ROWS_PER_W = N // NW
CHUNK = 128
NCHUNK = ROWS_PER_W // CHUNK
NDAY, NWEEK, NMONTH = 32, 7, 13
WM = NWEEK * NMONTH
COMB = NDAY * WM

_mesh = plsc.VectorSubcoreMesh(
    core_axis_name="c", subcore_axis_name="s", num_cores=NC, num_subcores=NS)


@functools.partial(
    pl.kernel,
    out_type=jax.ShapeDtypeStruct((COMB * D,), jnp.float32),
    mesh=_mesh,
    scratch_types=[
        pltpu.VMEM((D,), jnp.float32),
        pltpu.VMEM((NWEEK * D,), jnp.float32),
        pltpu.VMEM((NMONTH * D,), jnp.float32),
        pltpu.VMEM((WM * D,), jnp.float32),
    ],
)
def _build_comb(dt_h, wt_h, mt_h, comb_h, drow_v, wt_v, mt_v, buf_v):
    wid = lax.axis_index("s") * NC + lax.axis_index("c")
    pltpu.sync_copy(dt_h.at[pl.ds(wid * D, D)], drow_v)
    pltpu.sync_copy(wt_h, wt_v)
    pltpu.sync_copy(mt_h, mt_v)
    for w in range(NWEEK):
        dw = [drow_v[pl.ds(j * 16, 16)] + wt_v[pl.ds(w * D + j * 16, 16)]
              for j in range(D // 16)]
        for m in range(NMONTH):
            o = (w * NMONTH + m) * D
            for j in range(D // 16):
                buf_v[pl.ds(o + j * 16, 16)] = dw[j] + mt_v[pl.ds(m * D + j * 16, 16)]
    pltpu.sync_copy(buf_v, comb_h.at[pl.ds(wid * WM * D, WM * D)])


IDXBLK = 2560
NBLK = ROWS_PER_W // IDXBLK
GCH = 128
NCH = ROWS_PER_W // GCH


@functools.partial(
    pl.kernel,
    out_type=jax.ShapeDtypeStruct((N, D), jnp.float32),
    mesh=_mesh,
    scratch_types=[
        pltpu.VMEM((IDXBLK,), jnp.int32),
        pltpu.VMEM((IDXBLK,), jnp.int32),
        pltpu.VMEM((IDXBLK,), jnp.int32),
        pltpu.VMEM((ROWS_PER_W // 128, 128), jnp.int32),
        pltpu.VMEM((GCH, D), jnp.float32),
        pltpu.VMEM((GCH, D), jnp.float32),
        pltpu.VMEM((GCH, D), jnp.float32),
        pltpu.VMEM((GCH, D), jnp.float32),
        pltpu.SemaphoreType.DMA,
        pltpu.SemaphoreType.DMA,
        pltpu.SemaphoreType.DMA,
        pltpu.SemaphoreType.DMA,
        pltpu.SemaphoreType.DMA,
        pltpu.SemaphoreType.DMA,
        pltpu.SemaphoreType.DMA,
        pltpu.SemaphoreType.DMA,
    ],
)
def _lookup(day_h, week_h, month_h, comb_h, out_h,
            di, wi, mi, ci, rows0, rows1, rows2, rows3,
            gs0, gs1, gs2, gs3, ws0, ws1, ws2, ws3):
    wid = lax.axis_index("s") * NC + lax.axis_index("c")
    base0 = wid * ROWS_PER_W

    def blk(b, carry):
        boff = base0 + b * IDXBLK
        pltpu.sync_copy(day_h.at[pl.ds(boff, IDXBLK)], di)
        pltpu.sync_copy(week_h.at[pl.ds(boff, IDXBLK)], wi)
        pltpu.sync_copy(month_h.at[pl.ds(boff, IDXBLK)], mi)
        for g in range(IDXBLK // 16):
            s = pl.ds(g * 16, 16)
            d = jnp.clip(di[s], 0, NDAY - 1)
            w = jnp.clip(wi[s], 0, NWEEK - 1)
            m = jnp.clip(mi[s], 0, NMONTH - 1)
            ci[b * (IDXBLK // 128) + g // 8, pl.ds((g % 8) * 16, 16)] = (
                d * WM + w * NMONTH + m)
        return carry

    lax.fori_loop(0, NBLK, blk, 0)

    bufs = (rows0, rows1, rows2, rows3)
    gsems = (gs0, gs1, gs2, gs3)
    wsems = (ws0, ws1, ws2, ws3)

    def gather(c, rbuf, sem):
        return pltpu.make_async_copy(comb_h.at[ci.at[c]], rbuf, sem)

    def write(c, rbuf, sem):
        return pltpu.make_async_copy(
            rbuf, out_h.at[pl.ds(base0 + c * GCH, GCH)], sem)

    gather(0, bufs[0], gsems[0]).start()
    gather(1, bufs[1], gsems[1]).start()

    def step(q, carry):
        for j in range(4):
            c = 4 * q + j
            jn = (j + 2) % 4
            nc = c + 2
            gather(c, bufs[j], gsems[j]).wait()
            write(c, bufs[j], wsems[j]).start()

            @pl.when(nc < NCH)
            def _():
                @pl.when(nc >= 4)
                def _():
                    write(nc - 4, bufs[jn], wsems[jn]).wait()
                gather(nc, bufs[jn], gsems[jn]).start()
        return carry

    lax.fori_loop(0, NCH // 4, step, 0)
    for j in range(4):
        write(NCH - 4 + j, bufs[j], wsems[j]).wait()


def kernel(day, week, month, day_table, week_table, month_table):
    day = day.reshape(N).astype(jnp.int32)
    week = week.reshape(N).astype(jnp.int32)
    month = month.reshape(N).astype(jnp.int32)
    comb = _build_comb(day_table.reshape(-1), week_table.reshape(-1),
                       month_table.reshape(-1)).reshape(COMB, D)
    out = _lookup(day, week, month, comb)
    return out.reshape(B, L, D)


# speedup vs baseline: 35.3655x; 1.7040x over previous
import functools

import jax
import jax.numpy as jnp
from jax import lax
from jax.experimental import pallas as pl
from jax.experimental.pallas import tpu as pltpu
from jax.experimental.pallas import tpu_sc as plsc

NC, NS = 2, 16
NW = NC * NS
B, L, D = 4096, 200, 128
N = B * L
ROWS_PER_W = N // NW
CHUNK = 128
NCHUNK = ROWS_PER_W // CHUNK
NDAY, NWEEK, NMONTH = 32, 7, 13
WM = NWEEK * NMONTH
COMB = NDAY * WM
COMB_PAD = 2944

_mesh = plsc.VectorSubcoreMesh(
    core_axis_name="c", subcore_axis_name="s", num_cores=NC, num_subcores=NS)


@functools.partial(
    pl.kernel,
    out_type=jax.ShapeDtypeStruct((COMB_PAD * D,), jnp.float32),
    mesh=_mesh,
    scratch_types=[
        pltpu.VMEM((D,), jnp.float32),
        pltpu.VMEM((NWEEK * D,), jnp.float32),
        pltpu.VMEM((NMONTH * D,), jnp.float32),
        pltpu.VMEM((WM * D,), jnp.float32),
    ],
)
def _build_comb(dt_h, wt_h, mt_h, comb_h, drow_v, wt_v, mt_v, buf_v):
    wid = lax.axis_index("s") * NC + lax.axis_index("c")
    pltpu.sync_copy(dt_h.at[pl.ds(wid * D, D)], drow_v)
    pltpu.sync_copy(wt_h, wt_v)
    pltpu.sync_copy(mt_h, mt_v)
    for w in range(NWEEK):
        dw = [drow_v[pl.ds(j * 16, 16)] + wt_v[pl.ds(w * D + j * 16, 16)]
              for j in range(D // 16)]
        for m in range(NMONTH):
            o = (w * NMONTH + m) * D
            for j in range(D // 16):
                buf_v[pl.ds(o + j * 16, 16)] = dw[j] + mt_v[pl.ds(m * D + j * 16, 16)]
    pltpu.sync_copy(buf_v, comb_h.at[pl.ds(wid * WM * D, WM * D)])


IDXBLK = 2560
NBLK = ROWS_PER_W // IDXBLK
GCH = 128
NCH = ROWS_PER_W // GCH


@functools.partial(
    pl.kernel,
    out_type=jax.ShapeDtypeStruct((N, D), jnp.float32),
    mesh=_mesh,
    scratch_types=[
        pltpu.VMEM((IDXBLK,), jnp.int32),
        pltpu.VMEM((IDXBLK,), jnp.int32),
        pltpu.VMEM((IDXBLK,), jnp.int32),
        pltpu.VMEM((ROWS_PER_W // 128, 128), jnp.int32),
        pltpu.VMEM_SHARED((COMB_PAD, D), jnp.float32),
        pltpu.VMEM((GCH, D), jnp.float32),
        pltpu.VMEM((GCH, D), jnp.float32),
        pltpu.VMEM((GCH, D), jnp.float32),
        pltpu.VMEM((GCH, D), jnp.float32),
        pltpu.SemaphoreType.DMA,
        pltpu.SemaphoreType.DMA,
        pltpu.SemaphoreType.DMA,
        pltpu.SemaphoreType.DMA,
        pltpu.SemaphoreType.DMA,
        pltpu.SemaphoreType.DMA,
        pltpu.SemaphoreType.DMA,
        pltpu.SemaphoreType.DMA,
    ],
)
def _lookup(day_h, week_h, month_h, comb_h, out_h,
            di, wi, mi, ci, comb_sh, rows0, rows1, rows2, rows3,
            gs0, gs1, gs2, gs3, ws0, ws1, ws2, ws3):
    sid = lax.axis_index("s")
    wid = sid * NC + lax.axis_index("c")
    base0 = wid * ROWS_PER_W

    srows = COMB_PAD // NS
    pltpu.sync_copy(comb_h.at[pl.ds(sid * srows, srows)],
                    comb_sh.at[pl.ds(sid * srows, srows)])
    plsc.subcore_barrier()

    def blk(b, carry):
        boff = base0 + b * IDXBLK
        pltpu.sync_copy(day_h.at[pl.ds(boff, IDXBLK)], di)
        pltpu.sync_copy(week_h.at[pl.ds(boff, IDXBLK)], wi)
        pltpu.sync_copy(month_h.at[pl.ds(boff, IDXBLK)], mi)
        for g in range(IDXBLK // 16):
            s = pl.ds(g * 16, 16)
            d = jnp.clip(di[s], 0, NDAY - 1)
            w = jnp.clip(wi[s], 0, NWEEK - 1)
            m = jnp.clip(mi[s], 0, NMONTH - 1)
            ci[b * (IDXBLK // 128) + g // 8, pl.ds((g % 8) * 16, 16)] = (
                d * WM + w * NMONTH + m)
        return carry

    lax.fori_loop(0, NBLK, blk, 0)

    bufs = (rows0, rows1, rows2, rows3)
    gsems = (gs0, gs1, gs2, gs3)
    wsems = (ws0, ws1, ws2, ws3)

    def gather(c, rbuf, sem):
        return pltpu.make_async_copy(comb_sh.at[ci.at[c]], rbuf, sem)

    def write(c, rbuf, sem):
        return pltpu.make_async_copy(
            rbuf, out_h.at[pl.ds(base0 + c * GCH, GCH)], sem)

    gather(0, bufs[0], gsems[0]).start()
    gather(1, bufs[1], gsems[1]).start()

    def step(q, carry):
        for j in range(4):
            c = 4 * q + j
            jn = (j + 2) % 4
            nc = c + 2
            gather(c, bufs[j], gsems[j]).wait()
            write(c, bufs[j], wsems[j]).start()

            @pl.when(nc < NCH)
            def _():
                @pl.when(nc >= 4)
                def _():
                    write(nc - 4, bufs[jn], wsems[jn]).wait()
                gather(nc, bufs[jn], gsems[jn]).start()
        return carry

    lax.fori_loop(0, NCH // 4, step, 0)
    for j in range(4):
        write(NCH - 4 + j, bufs[j], wsems[j]).wait()


def kernel(day, week, month, day_table, week_table, month_table):
    day = day.reshape(N).astype(jnp.int32)
    week = week.reshape(N).astype(jnp.int32)
    month = month.reshape(N).astype(jnp.int32)
    comb = _build_comb(day_table.reshape(-1), week_table.reshape(-1),
                       month_table.reshape(-1)).reshape(COMB_PAD, D)
    out = _lookup(day, week, month, comb)
    return out.reshape(B, L, D)


# speedup vs baseline: 36.2216x; 1.0242x over previous
import functools

import jax
import jax.numpy as jnp
from jax import lax
from jax.experimental import pallas as pl
from jax.experimental.pallas import tpu as pltpu
from jax.experimental.pallas import tpu_sc as plsc

NC, NS = 2, 16
NW = NC * NS
B, L, D = 4096, 200, 128
N = B * L
ROWS_PER_W = N // NW
NDAY, NWEEK, NMONTH = 32, 7, 13
DSTRIDE = 96
COMB = NDAY * DSTRIDE
DPT = NDAY // NS
SLAB = DPT * DSTRIDE

NPASS = 2
RPP = ROWS_PER_W // NPASS
IDXBLK = 2560
NBLK = RPP // IDXBLK
GCH = 128
NCH = RPP // GCH

_mesh = plsc.VectorSubcoreMesh(
    core_axis_name="c", subcore_axis_name="s", num_cores=NC, num_subcores=NS)


@functools.partial(
    pl.kernel,
    out_type=jax.ShapeDtypeStruct((N, D), jnp.float32),
    mesh=_mesh,
    scratch_types=[
        pltpu.VMEM((DPT * D,), jnp.float32),
        pltpu.VMEM((NWEEK * D,), jnp.float32),
        pltpu.VMEM((NMONTH * D,), jnp.float32),
        pltpu.VMEM((DSTRIDE, D), jnp.float32),
        pltpu.VMEM_SHARED((COMB, D), jnp.float32),
        pltpu.VMEM((IDXBLK,), jnp.int32),
        pltpu.VMEM((IDXBLK,), jnp.int32),
        pltpu.VMEM((IDXBLK,), jnp.int32),
        pltpu.VMEM((RPP // 128, 128), jnp.int32),
        pltpu.VMEM((GCH, D), jnp.float32),
        pltpu.VMEM((GCH, D), jnp.float32),
        pltpu.VMEM((GCH, D), jnp.float32),
        pltpu.VMEM((GCH, D), jnp.float32),
        pltpu.SemaphoreType.DMA,
        pltpu.SemaphoreType.DMA,
        pltpu.SemaphoreType.DMA,
        pltpu.SemaphoreType.DMA,
        pltpu.SemaphoreType.DMA,
        pltpu.SemaphoreType.DMA,
        pltpu.SemaphoreType.DMA,
        pltpu.SemaphoreType.DMA,
    ],
)
def _temporal_embed(day_h, week_h, month_h, dt_h, wt_h, mt_h, out_h,
                    dtv, wtv, mtv, slab, comb_sh, di, wi, mi, ci,
                    rows0, rows1, rows2, rows3,
                    gs0, gs1, gs2, gs3, ws0, ws1, ws2, ws3):
    sid = lax.axis_index("s")
    wid = sid * NC + lax.axis_index("c")
    base0 = wid * ROWS_PER_W

    pltpu.sync_copy(dt_h.at[pl.ds(sid * (DPT * D), DPT * D)], dtv)
    pltpu.sync_copy(wt_h, wtv)
    pltpu.sync_copy(mt_h, mtv)
    for dd in range(DPT):
        for w in range(NWEEK):
            dw = [dtv[pl.ds(dd * D + j * 16, 16)] + wtv[pl.ds(w * D + j * 16, 16)]
                  for j in range(D // 16)]
            for m in range(NMONTH):
                r = w * NMONTH + m
                for j in range(D // 16):
                    slab[r, pl.ds(j * 16, 16)] = dw[j] + mtv[pl.ds(m * D + j * 16, 16)]
        pltpu.sync_copy(
            slab, comb_sh.at[pl.ds(sid * SLAB + dd * DSTRIDE, DSTRIDE)])
    plsc.subcore_barrier()

    bufs = (rows0, rows1, rows2, rows3)
    gsems = (gs0, gs1, gs2, gs3)
    wsems = (ws0, ws1, ws2, ws3)

    for h in range(NPASS):
        pbase = base0 + h * RPP

        def blk(b, carry):
            boff = pbase + b * IDXBLK
            pltpu.sync_copy(day_h.at[pl.ds(boff, IDXBLK)], di)
            pltpu.sync_copy(week_h.at[pl.ds(boff, IDXBLK)], wi)
            pltpu.sync_copy(month_h.at[pl.ds(boff, IDXBLK)], mi)

            def grp(g, c2):
                s = pl.ds(g * 16, 16)
                d = jnp.clip(di[s], 0, NDAY - 1)
                w = jnp.clip(wi[s], 0, NWEEK - 1)
                m = jnp.clip(mi[s], 0, NMONTH - 1)
                ci[b * (IDXBLK // 128) + g // 8, pl.ds((g % 8) * 16, 16)] = (
                    d * DSTRIDE + w * NMONTH + m)
                return c2

            lax.fori_loop(0, IDXBLK // 16, grp, 0)
            return carry

        lax.fori_loop(0, NBLK, blk, 0)

        def gather(c, rbuf, sem):
            return pltpu.make_async_copy(comb_sh.at[ci.at[c]], rbuf, sem)

        def write(c, rbuf, sem):
            return pltpu.make_async_copy(
                rbuf, out_h.at[pl.ds(pbase + c * GCH, GCH)], sem)

        gather(0, bufs[0], gsems[0]).start()
        gather(1, bufs[1], gsems[1]).start()

        def step(q, carry):
            for j in range(4):
                c = 4 * q + j
                jn = (j + 2) % 4
                nc = c + 2
                gather(c, bufs[j], gsems[j]).wait()
                write(c, bufs[j], wsems[j]).start()

                @pl.when(nc < NCH)
                def _():
                    @pl.when(nc >= 4)
                    def _():
                        write(nc - 4, bufs[jn], wsems[jn]).wait()
                    gather(nc, bufs[jn], gsems[jn]).start()
            return carry

        lax.fori_loop(0, NCH // 4, step, 0)
        for j in range(4):
            write(NCH - 4 + j, bufs[j], wsems[j]).wait()


def kernel(day, week, month, day_table, week_table, month_table):
    day = day.reshape(N).astype(jnp.int32)
    week = week.reshape(N).astype(jnp.int32)
    month = month.reshape(N).astype(jnp.int32)
    out = _temporal_embed(day, week, month, day_table.reshape(-1),
                          week_table.reshape(-1), month_table.reshape(-1))
    return out.reshape(B, L, D)


# speedup vs baseline: 40.1734x; 1.1091x over previous
import functools

import jax
import jax.numpy as jnp
from jax import lax
from jax.experimental import pallas as pl
from jax.experimental.pallas import tpu as pltpu
from jax.experimental.pallas import tpu_sc as plsc

NC, NS = 2, 16
NW = NC * NS
B, L, D = 4096, 200, 128
N = B * L
ROWS_PER_W = N // NW
NDAY, NWEEK, NMONTH = 32, 7, 13
DSTRIDE = 96
COMB = NDAY * DSTRIDE
DPT = NDAY // NS
SLAB = DPT * DSTRIDE

IDXBLK = 2560
NBLK = ROWS_PER_W // IDXBLK
GCH = 128
CPB = IDXBLK // GCH
NCH = ROWS_PER_W // GCH

_mesh = plsc.VectorSubcoreMesh(
    core_axis_name="c", subcore_axis_name="s", num_cores=NC, num_subcores=NS)


@functools.partial(
    pl.kernel,
    out_type=jax.ShapeDtypeStruct((N, D), jnp.float32),
    mesh=_mesh,
    scratch_types=[
        pltpu.VMEM((DPT * D,), jnp.float32),
        pltpu.VMEM((NWEEK * D,), jnp.float32),
        pltpu.VMEM((NMONTH * D,), jnp.float32),
        pltpu.VMEM((DSTRIDE, D), jnp.float32),
        pltpu.VMEM_SHARED((COMB, D), jnp.float32),
        pltpu.VMEM((2, IDXBLK), jnp.int32),
        pltpu.VMEM((2, IDXBLK), jnp.int32),
        pltpu.VMEM((2, IDXBLK), jnp.int32),
        pltpu.VMEM((2, CPB, 128), jnp.int32),
        pltpu.VMEM((GCH, D), jnp.float32),
        pltpu.VMEM((GCH, D), jnp.float32),
        pltpu.VMEM((GCH, D), jnp.float32),
        pltpu.VMEM((GCH, D), jnp.float32),
        pltpu.SemaphoreType.DMA,
        pltpu.SemaphoreType.DMA,
        pltpu.SemaphoreType.DMA,
        pltpu.SemaphoreType.DMA,
        pltpu.SemaphoreType.DMA,
        pltpu.SemaphoreType.DMA,
        pltpu.SemaphoreType.DMA,
        pltpu.SemaphoreType.DMA,
        pltpu.SemaphoreType.DMA,
        pltpu.SemaphoreType.DMA,
    ],
)
def _temporal_embed(day_h, week_h, month_h, dt_h, wt_h, mt_h, out_h,
                    dtv, wtv, mtv, slab, comb_sh, di, wi, mi, cbuf,
                    rows0, rows1, rows2, rows3, is0, is1,
                    gs0, gs1, gs2, gs3, ws0, ws1, ws2, ws3):
    sid = lax.axis_index("s")
    wid = sid * NC + lax.axis_index("c")
    base0 = wid * ROWS_PER_W

    pltpu.sync_copy(dt_h.at[pl.ds(sid * (DPT * D), DPT * D)], dtv)
    pltpu.sync_copy(wt_h, wtv)
    pltpu.sync_copy(mt_h, mtv)
    for dd in range(DPT):
        for w in range(NWEEK):
            dw = [dtv[pl.ds(dd * D + j * 16, 16)] + wtv[pl.ds(w * D + j * 16, 16)]
                  for j in range(D // 16)]
            for m in range(NMONTH):
                r = w * NMONTH + m
                for j in range(D // 16):
                    slab[r, pl.ds(j * 16, 16)] = dw[j] + mtv[pl.ds(m * D + j * 16, 16)]
        pltpu.sync_copy(
            slab, comb_sh.at[pl.ds(sid * SLAB + dd * DSTRIDE, DSTRIDE)])
    plsc.subcore_barrier()

    bufs = (rows0, rows1, rows2, rows3)
    gsems = (gs0, gs1, gs2, gs3)
    wsems = (ws0, ws1, ws2, ws3)
    isems = (is0, is1)

    def idx_copies(t, sl):
        boff = base0 + t * IDXBLK
        sem = isems[sl]
        return [
            pltpu.make_async_copy(day_h.at[pl.ds(boff, IDXBLK)], di.at[sl], sem),
            pltpu.make_async_copy(week_h.at[pl.ds(boff, IDXBLK)], wi.at[sl], sem),
            pltpu.make_async_copy(month_h.at[pl.ds(boff, IDXBLK)], mi.at[sl], sem),
        ]

    def compute_cidx(sl):
        def grp(g, c2):
            s = pl.ds(g * 16, 16)
            d = jnp.clip(di[sl, s], 0, NDAY - 1)
            w = jnp.clip(wi[sl, s], 0, NWEEK - 1)
            m = jnp.clip(mi[sl, s], 0, NMONTH - 1)
            cbuf[sl, g // 8, pl.ds((g % 8) * 16, 16)] = (
                d * DSTRIDE + w * NMONTH + m)
            return c2

        lax.fori_loop(0, IDXBLK // 16, grp, 0)

    def gather(t, cl, rbuf, sem):
        return pltpu.make_async_copy(
            comb_sh.at[cbuf.at[t % 2, cl]], rbuf, sem)

    def write(c, rbuf, sem):
        return pltpu.make_async_copy(
            rbuf, out_h.at[pl.ds(base0 + c * GCH, GCH)], sem)

    for cp in idx_copies(0, 0):
        cp.start()
    for cp in idx_copies(0, 0):
        cp.wait()
    compute_cidx(0)
    for cp in idx_copies(1, 1):
        cp.start()
    gather(0, 0, bufs[0], gsems[0]).start()
    gather(0, 1, bufs[1], gsems[1]).start()

    for t in range(NBLK):
        if t + 1 < NBLK:
            for cp in idx_copies(t + 1, (t + 1) % 2):
                cp.wait()
            compute_cidx((t + 1) % 2)
        if t + 2 < NBLK:
            for cp in idx_copies(t + 2, t % 2):
                cp.start()

        def step(q, carry):
            for j in range(4):
                cl = 4 * q + j
                c = t * CPB + cl
                jn = (j + 2) % 4
                gather(t, cl, bufs[j], gsems[j]).wait()
                write(c, bufs[j], wsems[j]).start()

                @pl.when(c + 2 >= 4)
                def _():
                    write(c - 2, bufs[jn], wsems[jn]).wait()
                gather(t, cl + 2, bufs[jn], gsems[jn]).start()
            return carry

        lax.fori_loop(0, 4, step, 0)

        for cl in (16, 17, 18, 19):
            jb = cl % 4
            c = t * CPB + cl
            gather(t, cl, bufs[jb], gsems[jb]).wait()
            write(c, bufs[jb], wsems[jb]).start()
            if cl < 18:
                jn = (jb + 2) % 4
                write(c - 2, bufs[jn], wsems[jn]).wait()
                gather(t, cl + 2, bufs[jn], gsems[jn]).start()
            elif t + 1 < NBLK:
                jn = (jb + 2) % 4
                write(c - 2, bufs[jn], wsems[jn]).wait()
                gather(t + 1, cl - 18, bufs[jn], gsems[jn]).start()

    for j in range(4):
        write(NCH - 4 + j, bufs[j], wsems[j]).wait()


def kernel(day, week, month, day_table, week_table, month_table):
    day = day.reshape(N).astype(jnp.int32)
    week = week.reshape(N).astype(jnp.int32)
    month = month.reshape(N).astype(jnp.int32)
    out = _temporal_embed(day, week, month, day_table.reshape(-1),
                          week_table.reshape(-1), month_table.reshape(-1))
    return out.reshape(B, L, D)
